# baseline (speedup 1.0000x reference)
import functools

import jax
import jax.numpy as jnp
from jax.experimental import pallas as pl
from jax.experimental.pallas import tpu as pltpu

V = 16
MAXN = 4 * V
B, K, H, W = 16, 128, 256, 256
C = 2 * V
HW = H * W
KG = 16
N_POLY = B * K
P2 = N_POLY // 2
P2_BLK = 128
N_BLK = P2 // P2_BLK


def _gather_body(ind_ref, *refs):
    ins = refs[:KG]
    out_ref = refs[KG]
    b = pl.program_id(0)
    kg = pl.program_id(1)
    base = (b * (K // KG) + kg) * KG
    lane = jax.lax.broadcasted_iota(jnp.int32, (1, 128), 1)
    z = jnp.zeros((1, 128), jnp.float32)
    rows = []
    for j in range(KG):
        r = ind_ref[base + j] % 128
        oh = (lane == r).astype(jnp.float32)
        rows.append(jnp.concatenate(
            [z] * j + [oh] + [z] * (KG - 1 - j), axis=1))
    ohm = jnp.concatenate(rows, axis=0)
    slabs = jnp.concatenate([ins[j][0] for j in range(KG)], axis=1)
    out_ref[...] = jax.lax.dot_general(
        ohm, slabs, (((1,), (1,)), ((), ())),
        preferred_element_type=jnp.float32)


def _gather(output3, ind_flat):
    in_specs = [
        pl.BlockSpec(
            (1, C, 128),
            functools.partial(
                lambda b, kg, ind, j=0: (b, 0, ind[(b * (K // KG) + kg) * KG + j] // 128),
                j=j))
        for j in range(KG)
    ]
    out_spec = pl.BlockSpec((KG, C), lambda b, kg, ind: (b * (K // KG) + kg, 0))
    grid_spec = pltpu.PrefetchScalarGridSpec(
        num_scalar_prefetch=1,
        grid=(B, K // KG),
        in_specs=in_specs,
        out_specs=out_spec,
    )
    return pl.pallas_call(
        _gather_body,
        grid_spec=grid_spec,
        out_shape=jax.ShapeDtypeStruct((N_POLY, C), jnp.float32),
        compiler_params=pltpu.CompilerParams(
            dimension_semantics=("parallel", "arbitrary")),
    )(ind_flat, *([output3] * KG))


def _shr1(a):
    return jnp.concatenate(
        [jnp.zeros((a.shape[0], 1), a.dtype), a[:, :-1]], axis=1)


def _shrk(a, k):
    return jnp.concatenate(
        [jnp.zeros((a.shape[0], k), a.dtype), a[:, :-k]], axis=1)


def _shl1(a):
    return jnp.concatenate(
        [a[:, 1:], jnp.zeros((a.shape[0], 1), a.dtype)], axis=1)


def _safe(d):
    return jnp.where(d == 0.0, jnp.ones_like(d), d)


def _shoelace_pair(x32, y32):
    xn = jnp.concatenate(
        [x32[:, 1:16], x32[:, :1], x32[:, 17:32], x32[:, 16:17]], axis=1)
    yn = jnp.concatenate(
        [y32[:, 1:16], y32[:, :1], y32[:, 17:32], y32[:, 16:17]], axis=1)
    tl = x32 * yn
    tr = y32 * xn
    lL = jnp.sum(tl[:, :16], axis=1, keepdims=True) + x32[:, :1] * y32[:, 1:2]
    rL = jnp.sum(tr[:, :16], axis=1, keepdims=True) + y32[:, :1] * x32[:, 1:2]
    lR = jnp.sum(tl[:, 16:], axis=1, keepdims=True) + x32[:, 16:17] * y32[:, 17:18]
    rR = jnp.sum(tr[:, 16:], axis=1, keepdims=True) + y32[:, 16:17] * x32[:, 17:18]
    return jnp.abs(0.5 * (rL - lL)), jnp.abs(0.5 * (rR - lR))


def _polyloss_body(pxy_ref, txy_ref, m_ref, out_ref):
    f32 = jnp.float32
    i16 = jax.lax.broadcasted_iota(jnp.int32, (P2_BLK, V), 1)
    a = pxy_ref[...]
    t = txy_ref[...]
    px2 = jnp.take_along_axis(a, 2 * i16, axis=1)
    py2 = jnp.take_along_axis(a, 2 * i16 + 1, axis=1)
    qx2 = jnp.take_along_axis(a, 2 * i16 + 2 * V, axis=1)
    qy2 = jnp.take_along_axis(a, 2 * i16 + 2 * V + 1, axis=1)
    tx2 = jnp.take_along_axis(t, 2 * i16, axis=1)
    ty2 = jnp.take_along_axis(t, 2 * i16 + 1, axis=1)
    ux2 = jnp.take_along_axis(t, 2 * i16 + 2 * V, axis=1)
    uy2 = jnp.take_along_axis(t, 2 * i16 + 2 * V + 1, axis=1)

    P = P2_BLK
    ii = jax.lax.broadcasted_iota(jnp.int32, (P, 2 * MAXN), 1)
    half = ii >= MAXN
    iota = ii.astype(f32)
    iotam = jnp.where(half, iota - float(MAXN), iota)
    zpad = jnp.zeros((P, MAXN - V), f32)
    x = jnp.concatenate([px2, zpad, qx2, zpad], axis=1)
    y = jnp.concatenate([py2, zpad, qy2, zpad], axis=1)
    nL = jnp.full((P, 1), float(V), f32)
    nR = jnp.full((P, 1), float(V), f32)
    base = jnp.where(half, MAXN, 0)

    for e in range(V):
        ep = (e - 1) % V
        c1x = jnp.where(half, ux2[:, ep:ep + 1], tx2[:, ep:ep + 1])
        c1y = jnp.where(half, uy2[:, ep:ep + 1], ty2[:, ep:ep + 1])
        c2x = jnp.where(half, ux2[:, e:e + 1], tx2[:, e:e + 1])
        c2y = jnp.where(half, uy2[:, e:e + 1], ty2[:, e:e + 1])
        n_b = jnp.where(half, nR, nL)

        idxlast = jnp.concatenate(
            [jnp.maximum(nL.astype(jnp.int32) - 1, 0),
             jnp.maximum(nR.astype(jnp.int32) - 1, 0) + MAXN], axis=1)
        lx = jnp.take_along_axis(x, idxlast, axis=1)
        ly = jnp.take_along_axis(y, idxlast, axis=1)
        pxv = jnp.where(ii == 0, lx[:, 0:1],
                        jnp.where(ii == MAXN, lx[:, 1:2], _shr1(x)))
        pyv = jnp.where(ii == 0, ly[:, 0:1],
                        jnp.where(ii == MAXN, ly[:, 1:2], _shr1(y)))

        valid = iotam < n_b
        ex, ey = c2x - c1x, c2y - c1y
        ins_c = (ex * (y - c1y) - ey * (x - c1x)) <= 0.0
        ins_p = (ex * (pyv - c1y) - ey * (pxv - c1x)) <= 0.0

        dx12 = x - pxv
        dy12 = y - pyv
        m1 = dy12 / _safe(dx12)
        b1 = pyv - m1 * pxv
        m2 = ey / _safe(ex)
        b2 = c1y - m2 * c1x
        x_gen = (b2 - b1) / _safe(m1 - m2)
        y_gen = m1 * x_gen + b1
        y_v1 = m2 * pxv + b2
        y_v2 = m1 * c1x + b1
        vert1 = dx12 == 0.0
        vert2 = ex == 0.0
        ix = jnp.where(vert1, pxv, jnp.where(vert2, c1x, x_gen))
        iy = jnp.where(vert1, y_v1, jnp.where(vert2, y_v2, y_gen))

        emit_i = valid & (ins_c != ins_p)
        emit_c = valid & ins_c
        cnt = emit_i.astype(f32) + emit_c.astype(f32)
        csum = cnt
        for s in (1, 2, 4, 8, 16, 32, 64):
            csum = csum + _shrk(csum, s)
        csum = jnp.where(half, csum - csum[:, MAXN - 1:MAXN], csum)
        totL = csum[:, MAXN - 1:MAXN]
        totR = csum[:, 2 * MAXN - 1:2 * MAXN]
        new_nL = jnp.minimum(totL, float(MAXN))
        new_nR = jnp.minimum(totR, float(MAXN))
        new_nb = jnp.where(half, new_nR, new_nL)

        def _sl(i):
            sL = csum[:, i:i + 1]
            sR = csum[:, MAXN + i:MAXN + i + 1]
            return jnp.where(half, sR, sL)

        t1 = _sl(31) <= iotam
        t2 = jnp.where(t1, _sl(47), _sl(15)) <= iotam
        cs3 = jnp.where(t1,
                        jnp.where(t2, _sl(55), _sl(39)),
                        jnp.where(t2, _sl(23), _sl(7)))
        t3 = cs3 <= iotam
        lo = (base + jnp.where(t1, 32, 0) + jnp.where(t2, 16, 0)
              + jnp.where(t3, 8, 0))
        for bit in (4, 2, 1):
            cand = lo + bit
            cs = jnp.take_along_axis(csum, cand - 1, axis=1)
            lo = jnp.where(cs <= iotam, cand, lo)

        lo_prev = jnp.concatenate(
            [jnp.full((P, 1), -1, jnp.int32), lo[:, :2 * MAXN - 1]], axis=1)
        is_first = lo > lo_prev
        fx = jnp.where(emit_i, ix, x)
        fy = jnp.where(emit_i, iy, y)
        g_fx = jnp.take_along_axis(fx, lo, axis=1)
        g_fy = jnp.take_along_axis(fy, lo, axis=1)
        g_cx = jnp.take_along_axis(x, lo, axis=1)
        g_cy = jnp.take_along_axis(y, lo, axis=1)
        keep = iotam < new_nb
        x = jnp.where(keep, jnp.where(is_first, g_fx, g_cx), 0.0)
        y = jnp.where(keep, jnp.where(is_first, g_fy, g_cy), 0.0)
        nL, nR = new_nL, new_nR

    n_b = jnp.where(half, nR, nL)
    valid_f = (iotam < n_b).astype(f32)
    is_last = iotam == (n_b - 1.0)
    x0b = jnp.where(half, x[:, MAXN:MAXN + 1], x[:, :1])
    y0b = jnp.where(half, y[:, MAXN:MAXN + 1], y[:, :1])
    x_nxt = jnp.where(is_last, x0b, _shl1(x))
    y_nxt = jnp.where(is_last, y0b, _shl1(y))
    tl = valid_f * x * y_nxt
    tr = valid_f * y * x_nxt
    hf = jnp.where(half, 1.0, 0.0).astype(f32)
    lL = jnp.sum(tl * (1.0 - hf), axis=1, keepdims=True)
    lR = jnp.sum(tl * hf, axis=1, keepdims=True)
    rL = jnp.sum(tr * (1.0 - hf), axis=1, keepdims=True)
    rR = jnp.sum(tr * hf, axis=1, keepdims=True)
    hasL = (nL > 0.0).astype(f32)
    hasR = (nR > 0.0).astype(f32)
    y1L = jnp.where(nL == 1.0, y[:, :1], y[:, 1:2])
    x1L = jnp.where(nL == 1.0, x[:, :1], x[:, 1:2])
    y1R = jnp.where(nR == 1.0, y[:, MAXN:MAXN + 1], y[:, MAXN + 1:MAXN + 2])
    x1R = jnp.where(nR == 1.0, x[:, MAXN:MAXN + 1], x[:, MAXN + 1:MAXN + 2])
    a_iL = jnp.abs(0.5 * ((rL + hasL * y[:, :1] * x1L) - (lL + hasL * x[:, :1] * y1L)))
    a_iR = jnp.abs(0.5 * ((rR + hasR * y[:, MAXN:MAXN + 1] * x1R)
                          - (lR + hasR * x[:, MAXN:MAXN + 1] * y1R)))

    x32p = jnp.concatenate([px2, qx2], axis=1)
    y32p = jnp.concatenate([py2, qy2], axis=1)
    x32t = jnp.concatenate([tx2, ux2], axis=1)
    y32t = jnp.concatenate([ty2, uy2], axis=1)
    a_pL, a_pR = _shoelace_pair(x32p, y32p)
    a_gL, a_gR = _shoelace_pair(x32t, y32t)

    intL = jnp.where(a_iL == 0.0, jnp.minimum(a_pL, a_gL), 0.0) + a_iL
    intR = jnp.where(a_iR == 0.0, jnp.minimum(a_pR, a_gR), 0.0) + a_iR
    iouL = intL / (a_gL + a_pL - intL + 1e-6)
    iouR = intR / (a_gR + a_pR - intR + 1e-6)

    s_iou = jnp.sum(iouL * m_ref[:, 0:1] + iouR * m_ref[:, 1:2])
    s_msk = jnp.sum(m_ref[...])
    li = jax.lax.broadcasted_iota(jnp.int32, (1, 128), 1)
    out_ref[0] = jnp.where(li == 0, s_iou, jnp.where(li == 1, s_msk, 0.0))


def _polyloss(pxy2, txy2, mask2):
    spec64 = pl.BlockSpec((P2_BLK, 4 * V), lambda g: (g, 0))
    spec2 = pl.BlockSpec((P2_BLK, 2), lambda g: (g, 0))
    return pl.pallas_call(
        _polyloss_body,
        grid=(N_BLK,),
        in_specs=[spec64, spec64, spec2],
        out_specs=pl.BlockSpec((1, 1, 128), lambda g: (g, 0, 0)),
        out_shape=jax.ShapeDtypeStruct((N_BLK, 1, 128), jnp.float32),
        compiler_params=pltpu.CompilerParams(
            dimension_semantics=("parallel",)),
    )(pxy2, txy2, mask2)


def kernel(output, mask, ind, target):
    output3 = output.reshape(B, C, HW)
    pred = _gather(output3, ind.reshape(-1))
    pxy2 = pred.reshape(P2, 2 * C)
    txy2 = target.reshape(P2, 2 * C)
    partials = _polyloss(pxy2, txy2, mask.reshape(P2, 2))
    s_iou = jnp.sum(partials[:, 0, 0])
    s_msk = jnp.sum(partials[:, 0, 1])
    return 1.0 - s_iou / (s_msk + 1e-6)


# speedup vs baseline: 1.0336x; 1.0336x over previous
import functools

import jax
import jax.numpy as jnp
from jax.experimental import pallas as pl
from jax.experimental.pallas import tpu as pltpu

V = 16
MAXN = 4 * V
B, K, H, W = 16, 128, 256, 256
C = 2 * V
HW = H * W
KG = 16
N_POLY = B * K
P2 = N_POLY // 2
P2_BLK = 128
N_BLK = P2 // P2_BLK


def _gather_body(ind_ref, *refs):
    ins = refs[:KG]
    out_ref = refs[KG]
    b = pl.program_id(0)
    kg = pl.program_id(1)
    base = (b * (K // KG) + kg) * KG
    lane = jax.lax.broadcasted_iota(jnp.int32, (1, 128), 1)
    z = jnp.zeros((1, 128), jnp.float32)
    rows = []
    for j in range(KG):
        r = ind_ref[base + j] % 128
        oh = (lane == r).astype(jnp.float32)
        rows.append(jnp.concatenate(
            [z] * j + [oh] + [z] * (KG - 1 - j), axis=1))
    ohm = jnp.concatenate(rows, axis=0)
    slabs = jnp.concatenate([ins[j][0] for j in range(KG)], axis=1)
    out_ref[...] = jax.lax.dot_general(
        ohm, slabs, (((1,), (1,)), ((), ())),
        preferred_element_type=jnp.float32)


def _gather(output3, ind_flat):
    in_specs = [
        pl.BlockSpec(
            (1, C, 128),
            functools.partial(
                lambda b, kg, ind, j=0: (b, 0, ind[(b * (K // KG) + kg) * KG + j] // 128),
                j=j))
        for j in range(KG)
    ]
    out_spec = pl.BlockSpec((KG, C), lambda b, kg, ind: (b * (K // KG) + kg, 0))
    grid_spec = pltpu.PrefetchScalarGridSpec(
        num_scalar_prefetch=1,
        grid=(B, K // KG),
        in_specs=in_specs,
        out_specs=out_spec,
    )
    return pl.pallas_call(
        _gather_body,
        grid_spec=grid_spec,
        out_shape=jax.ShapeDtypeStruct((N_POLY, C), jnp.float32),
        compiler_params=pltpu.CompilerParams(
            dimension_semantics=("arbitrary", "arbitrary")),
    )(ind_flat, *([output3] * KG))


def _shr1(a):
    return jnp.concatenate(
        [jnp.zeros((a.shape[0], 1), a.dtype), a[:, :-1]], axis=1)


def _shrk(a, k):
    return jnp.concatenate(
        [jnp.zeros((a.shape[0], k), a.dtype), a[:, :-k]], axis=1)


def _shl1(a):
    return jnp.concatenate(
        [a[:, 1:], jnp.zeros((a.shape[0], 1), a.dtype)], axis=1)


def _safe(d):
    return jnp.where(d == 0.0, jnp.ones_like(d), d)


def _shoelace_pair(x32, y32):
    xn = jnp.concatenate(
        [x32[:, 1:16], x32[:, :1], x32[:, 17:32], x32[:, 16:17]], axis=1)
    yn = jnp.concatenate(
        [y32[:, 1:16], y32[:, :1], y32[:, 17:32], y32[:, 16:17]], axis=1)
    tl = x32 * yn
    tr = y32 * xn
    lL = jnp.sum(tl[:, :16], axis=1, keepdims=True) + x32[:, :1] * y32[:, 1:2]
    rL = jnp.sum(tr[:, :16], axis=1, keepdims=True) + y32[:, :1] * x32[:, 1:2]
    lR = jnp.sum(tl[:, 16:], axis=1, keepdims=True) + x32[:, 16:17] * y32[:, 17:18]
    rR = jnp.sum(tr[:, 16:], axis=1, keepdims=True) + y32[:, 16:17] * x32[:, 17:18]
    return jnp.abs(0.5 * (rL - lL)), jnp.abs(0.5 * (rR - lR))


def _polyloss_body(pxy_ref, txy_ref, m_ref, out_ref):
    f32 = jnp.float32
    i16 = jax.lax.broadcasted_iota(jnp.int32, (P2_BLK, V), 1)
    a = pxy_ref[...]
    t = txy_ref[...]
    px2 = jnp.take_along_axis(a, 2 * i16, axis=1)
    py2 = jnp.take_along_axis(a, 2 * i16 + 1, axis=1)
    qx2 = jnp.take_along_axis(a, 2 * i16 + 2 * V, axis=1)
    qy2 = jnp.take_along_axis(a, 2 * i16 + 2 * V + 1, axis=1)
    tx2 = jnp.take_along_axis(t, 2 * i16, axis=1)
    ty2 = jnp.take_along_axis(t, 2 * i16 + 1, axis=1)
    ux2 = jnp.take_along_axis(t, 2 * i16 + 2 * V, axis=1)
    uy2 = jnp.take_along_axis(t, 2 * i16 + 2 * V + 1, axis=1)

    P = P2_BLK
    ii = jax.lax.broadcasted_iota(jnp.int32, (P, 2 * MAXN), 1)
    half = ii >= MAXN
    iota = ii.astype(f32)
    iotam = jnp.where(half, iota - float(MAXN), iota)
    zpad = jnp.zeros((P, MAXN - V), f32)
    x = jnp.concatenate([px2, zpad, qx2, zpad], axis=1)
    y = jnp.concatenate([py2, zpad, qy2, zpad], axis=1)
    nL = jnp.full((P, 1), float(V), f32)
    nR = jnp.full((P, 1), float(V), f32)
    base = jnp.where(half, MAXN, 0)

    for e in range(V):
        ep = (e - 1) % V
        c1x = jnp.where(half, ux2[:, ep:ep + 1], tx2[:, ep:ep + 1])
        c1y = jnp.where(half, uy2[:, ep:ep + 1], ty2[:, ep:ep + 1])
        c2x = jnp.where(half, ux2[:, e:e + 1], tx2[:, e:e + 1])
        c2y = jnp.where(half, uy2[:, e:e + 1], ty2[:, e:e + 1])
        n_b = jnp.where(half, nR, nL)

        idxlast = jnp.concatenate(
            [jnp.maximum(nL.astype(jnp.int32) - 1, 0),
             jnp.maximum(nR.astype(jnp.int32) - 1, 0) + MAXN], axis=1)
        lx = jnp.take_along_axis(x, idxlast, axis=1)
        ly = jnp.take_along_axis(y, idxlast, axis=1)
        pxv = jnp.where(ii == 0, lx[:, 0:1],
                        jnp.where(ii == MAXN, lx[:, 1:2], _shr1(x)))
        pyv = jnp.where(ii == 0, ly[:, 0:1],
                        jnp.where(ii == MAXN, ly[:, 1:2], _shr1(y)))

        valid = iotam < n_b
        ex, ey = c2x - c1x, c2y - c1y
        ins_c = (ex * (y - c1y) - ey * (x - c1x)) <= 0.0
        ins_p = (ex * (pyv - c1y) - ey * (pxv - c1x)) <= 0.0

        dx12 = x - pxv
        dy12 = y - pyv
        m1 = dy12 / _safe(dx12)
        b1 = pyv - m1 * pxv
        m2 = ey / _safe(ex)
        b2 = c1y - m2 * c1x
        x_gen = (b2 - b1) / _safe(m1 - m2)
        y_gen = m1 * x_gen + b1
        y_v1 = m2 * pxv + b2
        y_v2 = m1 * c1x + b1
        vert1 = dx12 == 0.0
        vert2 = ex == 0.0
        ix = jnp.where(vert1, pxv, jnp.where(vert2, c1x, x_gen))
        iy = jnp.where(vert1, y_v1, jnp.where(vert2, y_v2, y_gen))

        emit_i = valid & (ins_c != ins_p)
        emit_c = valid & ins_c
        cnt = emit_i.astype(f32) + emit_c.astype(f32)
        csum = cnt
        for s in (1, 2, 4, 8, 16, 32, 64):
            csum = csum + _shrk(csum, s)
        csum = jnp.where(half, csum - csum[:, MAXN - 1:MAXN], csum)
        totL = csum[:, MAXN - 1:MAXN]
        totR = csum[:, 2 * MAXN - 1:2 * MAXN]
        new_nL = jnp.minimum(totL, float(MAXN))
        new_nR = jnp.minimum(totR, float(MAXN))
        new_nb = jnp.where(half, new_nR, new_nL)

        lo = base
        for bit in (32, 16, 8, 4, 2, 1):
            cand = lo + bit
            cs = jnp.take_along_axis(csum, cand - 1, axis=1)
            lo = jnp.where(cs <= iotam, cand, lo)

        lo_prev = jnp.concatenate(
            [jnp.full((P, 1), -1, jnp.int32), lo[:, :2 * MAXN - 1]], axis=1)
        is_first = lo > lo_prev
        fx = jnp.where(emit_i, ix, x)
        fy = jnp.where(emit_i, iy, y)
        g_fx = jnp.take_along_axis(fx, lo, axis=1)
        g_fy = jnp.take_along_axis(fy, lo, axis=1)
        g_cx = jnp.take_along_axis(x, lo, axis=1)
        g_cy = jnp.take_along_axis(y, lo, axis=1)
        keep = iotam < new_nb
        x = jnp.where(keep, jnp.where(is_first, g_fx, g_cx), 0.0)
        y = jnp.where(keep, jnp.where(is_first, g_fy, g_cy), 0.0)
        nL, nR = new_nL, new_nR

    n_b = jnp.where(half, nR, nL)
    valid_f = (iotam < n_b).astype(f32)
    is_last = iotam == (n_b - 1.0)
    x0b = jnp.where(half, x[:, MAXN:MAXN + 1], x[:, :1])
    y0b = jnp.where(half, y[:, MAXN:MAXN + 1], y[:, :1])
    x_nxt = jnp.where(is_last, x0b, _shl1(x))
    y_nxt = jnp.where(is_last, y0b, _shl1(y))
    tl = valid_f * x * y_nxt
    tr = valid_f * y * x_nxt
    hf = jnp.where(half, 1.0, 0.0).astype(f32)
    lL = jnp.sum(tl * (1.0 - hf), axis=1, keepdims=True)
    lR = jnp.sum(tl * hf, axis=1, keepdims=True)
    rL = jnp.sum(tr * (1.0 - hf), axis=1, keepdims=True)
    rR = jnp.sum(tr * hf, axis=1, keepdims=True)
    hasL = (nL > 0.0).astype(f32)
    hasR = (nR > 0.0).astype(f32)
    y1L = jnp.where(nL == 1.0, y[:, :1], y[:, 1:2])
    x1L = jnp.where(nL == 1.0, x[:, :1], x[:, 1:2])
    y1R = jnp.where(nR == 1.0, y[:, MAXN:MAXN + 1], y[:, MAXN + 1:MAXN + 2])
    x1R = jnp.where(nR == 1.0, x[:, MAXN:MAXN + 1], x[:, MAXN + 1:MAXN + 2])
    a_iL = jnp.abs(0.5 * ((rL + hasL * y[:, :1] * x1L) - (lL + hasL * x[:, :1] * y1L)))
    a_iR = jnp.abs(0.5 * ((rR + hasR * y[:, MAXN:MAXN + 1] * x1R)
                          - (lR + hasR * x[:, MAXN:MAXN + 1] * y1R)))

    x32p = jnp.concatenate([px2, qx2], axis=1)
    y32p = jnp.concatenate([py2, qy2], axis=1)
    x32t = jnp.concatenate([tx2, ux2], axis=1)
    y32t = jnp.concatenate([ty2, uy2], axis=1)
    a_pL, a_pR = _shoelace_pair(x32p, y32p)
    a_gL, a_gR = _shoelace_pair(x32t, y32t)

    intL = jnp.where(a_iL == 0.0, jnp.minimum(a_pL, a_gL), 0.0) + a_iL
    intR = jnp.where(a_iR == 0.0, jnp.minimum(a_pR, a_gR), 0.0) + a_iR
    iouL = intL / (a_gL + a_pL - intL + 1e-6)
    iouR = intR / (a_gR + a_pR - intR + 1e-6)

    s_iou = jnp.sum(iouL * m_ref[:, 0:1] + iouR * m_ref[:, 1:2])
    s_msk = jnp.sum(m_ref[...])
    li = jax.lax.broadcasted_iota(jnp.int32, (1, 128), 1)
    out_ref[0] = jnp.where(li == 0, s_iou, jnp.where(li == 1, s_msk, 0.0))


def _polyloss(pxy2, txy2, mask2):
    spec64 = pl.BlockSpec((P2_BLK, 4 * V), lambda g: (g, 0))
    spec2 = pl.BlockSpec((P2_BLK, 2), lambda g: (g, 0))
    return pl.pallas_call(
        _polyloss_body,
        grid=(N_BLK,),
        in_specs=[spec64, spec64, spec2],
        out_specs=pl.BlockSpec((1, 1, 128), lambda g: (g, 0, 0)),
        out_shape=jax.ShapeDtypeStruct((N_BLK, 1, 128), jnp.float32),
        compiler_params=pltpu.CompilerParams(
            dimension_semantics=("arbitrary",)),
    )(pxy2, txy2, mask2)


def kernel(output, mask, ind, target):
    output3 = output.reshape(B, C, HW)
    pred = _gather(output3, ind.reshape(-1))
    pxy2 = pred.reshape(P2, 2 * C)
    txy2 = target.reshape(P2, 2 * C)
    partials = _polyloss(pxy2, txy2, mask.reshape(P2, 2))
    s_iou = jnp.sum(partials[:, 0, 0])
    s_msk = jnp.sum(partials[:, 0, 1])
    return 1.0 - s_iou / (s_msk + 1e-6)


# speedup vs baseline: 1.1015x; 1.0658x over previous
import functools

import jax
import jax.numpy as jnp
from jax.experimental import pallas as pl
from jax.experimental.pallas import tpu as pltpu

V = 16
MAXN = 4 * V
B, K, H, W = 16, 128, 256, 256
C = 2 * V
HW = H * W
KG = 16
N_POLY = B * K
P2 = N_POLY // 2
P2_BLK = 128
N_BLK = P2 // P2_BLK


def _gather_body(x_ref, iw_ref, ic_ref, out_ref):
    f32 = jnp.float32
    iw = iw_ref[0] // 128
    ic = ic_ref[0] % 128
    sub512 = jax.lax.broadcasted_iota(jnp.int32, (HW // 128, K), 0)
    ow = (sub512 == iw).astype(f32)
    lane = jax.lax.broadcasted_iota(jnp.int32, (K, 128), 1)
    ohr = (lane == ic).astype(f32)
    cols = []
    for c in range(C):
        w1 = jax.lax.dot_general(
            ow, x_ref[0, c], (((0,), (0,)), ((), ())),
            preferred_element_type=f32)
        cols.append(jnp.sum(w1 * ohr, axis=1, keepdims=True))
    out_ref[...] = jnp.concatenate(cols, axis=1)


def _gather(output4, ind3, indc):
    return pl.pallas_call(
        _gather_body,
        grid=(B,),
        in_specs=[
            pl.BlockSpec((1, C, HW // 128, 128), lambda b: (b, 0, 0, 0)),
            pl.BlockSpec((1, 1, K), lambda b: (b, 0, 0)),
            pl.BlockSpec((1, K, 1), lambda b: (b, 0, 0)),
        ],
        out_specs=pl.BlockSpec((K, C), lambda b: (b, 0)),
        out_shape=jax.ShapeDtypeStruct((N_POLY, C), jnp.float32),
        compiler_params=pltpu.CompilerParams(
            dimension_semantics=("arbitrary",)),
    )(output4, ind3, indc)


def _shr1(a):
    return jnp.concatenate(
        [jnp.zeros((a.shape[0], 1), a.dtype), a[:, :-1]], axis=1)


def _shrk(a, k):
    return jnp.concatenate(
        [jnp.zeros((a.shape[0], k), a.dtype), a[:, :-k]], axis=1)


def _shl1(a):
    return jnp.concatenate(
        [a[:, 1:], jnp.zeros((a.shape[0], 1), a.dtype)], axis=1)


def _safe(d):
    return jnp.where(d == 0.0, jnp.ones_like(d), d)


def _shoelace_pair(x32, y32):
    xn = jnp.concatenate(
        [x32[:, 1:16], x32[:, :1], x32[:, 17:32], x32[:, 16:17]], axis=1)
    yn = jnp.concatenate(
        [y32[:, 1:16], y32[:, :1], y32[:, 17:32], y32[:, 16:17]], axis=1)
    tl = x32 * yn
    tr = y32 * xn
    lL = jnp.sum(tl[:, :16], axis=1, keepdims=True) + x32[:, :1] * y32[:, 1:2]
    rL = jnp.sum(tr[:, :16], axis=1, keepdims=True) + y32[:, :1] * x32[:, 1:2]
    lR = jnp.sum(tl[:, 16:], axis=1, keepdims=True) + x32[:, 16:17] * y32[:, 17:18]
    rR = jnp.sum(tr[:, 16:], axis=1, keepdims=True) + y32[:, 16:17] * x32[:, 17:18]
    return jnp.abs(0.5 * (rL - lL)), jnp.abs(0.5 * (rR - lR))


def _polyloss_body(pxy_ref, txy_ref, m_ref, out_ref):
    f32 = jnp.float32
    i16 = jax.lax.broadcasted_iota(jnp.int32, (P2_BLK, V), 1)
    a = pxy_ref[...]
    t = txy_ref[...]
    px2 = jnp.take_along_axis(a, 2 * i16, axis=1)
    py2 = jnp.take_along_axis(a, 2 * i16 + 1, axis=1)
    qx2 = jnp.take_along_axis(a, 2 * i16 + 2 * V, axis=1)
    qy2 = jnp.take_along_axis(a, 2 * i16 + 2 * V + 1, axis=1)
    tx2 = jnp.take_along_axis(t, 2 * i16, axis=1)
    ty2 = jnp.take_along_axis(t, 2 * i16 + 1, axis=1)
    ux2 = jnp.take_along_axis(t, 2 * i16 + 2 * V, axis=1)
    uy2 = jnp.take_along_axis(t, 2 * i16 + 2 * V + 1, axis=1)

    P = P2_BLK
    ii = jax.lax.broadcasted_iota(jnp.int32, (P, 2 * MAXN), 1)
    half = ii >= MAXN
    iota = ii.astype(f32)
    iotam = jnp.where(half, iota - float(MAXN), iota)
    zpad = jnp.zeros((P, MAXN - V), f32)
    x = jnp.concatenate([px2, zpad, qx2, zpad], axis=1)
    y = jnp.concatenate([py2, zpad, qy2, zpad], axis=1)
    nL = jnp.full((P, 1), float(V), f32)
    nR = jnp.full((P, 1), float(V), f32)
    base = jnp.where(half, MAXN, 0)

    for e in range(V):
        ep = (e - 1) % V
        c1x = jnp.where(half, ux2[:, ep:ep + 1], tx2[:, ep:ep + 1])
        c1y = jnp.where(half, uy2[:, ep:ep + 1], ty2[:, ep:ep + 1])
        c2x = jnp.where(half, ux2[:, e:e + 1], tx2[:, e:e + 1])
        c2y = jnp.where(half, uy2[:, e:e + 1], ty2[:, e:e + 1])
        n_b = jnp.where(half, nR, nL)

        idxlast = jnp.concatenate(
            [jnp.maximum(nL.astype(jnp.int32) - 1, 0),
             jnp.maximum(nR.astype(jnp.int32) - 1, 0) + MAXN], axis=1)
        lx = jnp.take_along_axis(x, idxlast, axis=1)
        ly = jnp.take_along_axis(y, idxlast, axis=1)
        pxv = jnp.where(ii == 0, lx[:, 0:1],
                        jnp.where(ii == MAXN, lx[:, 1:2], _shr1(x)))
        pyv = jnp.where(ii == 0, ly[:, 0:1],
                        jnp.where(ii == MAXN, ly[:, 1:2], _shr1(y)))

        valid = iotam < n_b
        ex, ey = c2x - c1x, c2y - c1y
        ins_c = (ex * (y - c1y) - ey * (x - c1x)) <= 0.0
        ins_p = (ex * (pyv - c1y) - ey * (pxv - c1x)) <= 0.0

        dx12 = x - pxv
        dy12 = y - pyv
        m1 = dy12 / _safe(dx12)
        b1 = pyv - m1 * pxv
        m2 = ey / _safe(ex)
        b2 = c1y - m2 * c1x
        x_gen = (b2 - b1) / _safe(m1 - m2)
        y_gen = m1 * x_gen + b1
        y_v1 = m2 * pxv + b2
        y_v2 = m1 * c1x + b1
        vert1 = dx12 == 0.0
        vert2 = ex == 0.0
        ix = jnp.where(vert1, pxv, jnp.where(vert2, c1x, x_gen))
        iy = jnp.where(vert1, y_v1, jnp.where(vert2, y_v2, y_gen))

        emit_i = valid & (ins_c != ins_p)
        emit_c = valid & ins_c
        cnt = emit_i.astype(f32) + emit_c.astype(f32)
        csum = cnt
        for s in (1, 2, 4, 8, 16, 32, 64):
            csum = csum + _shrk(csum, s)
        csum = jnp.where(half, csum - csum[:, MAXN - 1:MAXN], csum)
        totL = csum[:, MAXN - 1:MAXN]
        totR = csum[:, 2 * MAXN - 1:2 * MAXN]
        new_nL = jnp.minimum(totL, float(MAXN))
        new_nR = jnp.minimum(totR, float(MAXN))
        new_nb = jnp.where(half, new_nR, new_nL)

        lo = base
        for bit in (32, 16, 8, 4, 2, 1):
            cand = lo + bit
            cs = jnp.take_along_axis(csum, cand - 1, axis=1)
            lo = jnp.where(cs <= iotam, cand, lo)

        lo_prev = jnp.concatenate(
            [jnp.full((P, 1), -1, jnp.int32), lo[:, :2 * MAXN - 1]], axis=1)
        is_first = lo > lo_prev
        fx = jnp.where(emit_i, ix, x)
        fy = jnp.where(emit_i, iy, y)
        g_fx = jnp.take_along_axis(fx, lo, axis=1)
        g_fy = jnp.take_along_axis(fy, lo, axis=1)
        g_cx = jnp.take_along_axis(x, lo, axis=1)
        g_cy = jnp.take_along_axis(y, lo, axis=1)
        keep = iotam < new_nb
        x = jnp.where(keep, jnp.where(is_first, g_fx, g_cx), 0.0)
        y = jnp.where(keep, jnp.where(is_first, g_fy, g_cy), 0.0)
        nL, nR = new_nL, new_nR

    n_b = jnp.where(half, nR, nL)
    valid_f = (iotam < n_b).astype(f32)
    is_last = iotam == (n_b - 1.0)
    x0b = jnp.where(half, x[:, MAXN:MAXN + 1], x[:, :1])
    y0b = jnp.where(half, y[:, MAXN:MAXN + 1], y[:, :1])
    x_nxt = jnp.where(is_last, x0b, _shl1(x))
    y_nxt = jnp.where(is_last, y0b, _shl1(y))
    tl = valid_f * x * y_nxt
    tr = valid_f * y * x_nxt
    hf = jnp.where(half, 1.0, 0.0).astype(f32)
    lL = jnp.sum(tl * (1.0 - hf), axis=1, keepdims=True)
    lR = jnp.sum(tl * hf, axis=1, keepdims=True)
    rL = jnp.sum(tr * (1.0 - hf), axis=1, keepdims=True)
    rR = jnp.sum(tr * hf, axis=1, keepdims=True)
    hasL = (nL > 0.0).astype(f32)
    hasR = (nR > 0.0).astype(f32)
    y1L = jnp.where(nL == 1.0, y[:, :1], y[:, 1:2])
    x1L = jnp.where(nL == 1.0, x[:, :1], x[:, 1:2])
    y1R = jnp.where(nR == 1.0, y[:, MAXN:MAXN + 1], y[:, MAXN + 1:MAXN + 2])
    x1R = jnp.where(nR == 1.0, x[:, MAXN:MAXN + 1], x[:, MAXN + 1:MAXN + 2])
    a_iL = jnp.abs(0.5 * ((rL + hasL * y[:, :1] * x1L) - (lL + hasL * x[:, :1] * y1L)))
    a_iR = jnp.abs(0.5 * ((rR + hasR * y[:, MAXN:MAXN + 1] * x1R)
                          - (lR + hasR * x[:, MAXN:MAXN + 1] * y1R)))

    x32p = jnp.concatenate([px2, qx2], axis=1)
    y32p = jnp.concatenate([py2, qy2], axis=1)
    x32t = jnp.concatenate([tx2, ux2], axis=1)
    y32t = jnp.concatenate([ty2, uy2], axis=1)
    a_pL, a_pR = _shoelace_pair(x32p, y32p)
    a_gL, a_gR = _shoelace_pair(x32t, y32t)

    intL = jnp.where(a_iL == 0.0, jnp.minimum(a_pL, a_gL), 0.0) + a_iL
    intR = jnp.where(a_iR == 0.0, jnp.minimum(a_pR, a_gR), 0.0) + a_iR
    iouL = intL / (a_gL + a_pL - intL + 1e-6)
    iouR = intR / (a_gR + a_pR - intR + 1e-6)

    s_iou = jnp.sum(iouL * m_ref[:, 0:1] + iouR * m_ref[:, 1:2])
    s_msk = jnp.sum(m_ref[...])
    li = jax.lax.broadcasted_iota(jnp.int32, (1, 128), 1)
    out_ref[0] = jnp.where(li == 0, s_iou, jnp.where(li == 1, s_msk, 0.0))


def _polyloss(pxy2, txy2, mask2):
    spec64 = pl.BlockSpec((P2_BLK, 4 * V), lambda g: (g, 0))
    spec2 = pl.BlockSpec((P2_BLK, 2), lambda g: (g, 0))
    return pl.pallas_call(
        _polyloss_body,
        grid=(N_BLK,),
        in_specs=[spec64, spec64, spec2],
        out_specs=pl.BlockSpec((1, 1, 128), lambda g: (g, 0, 0)),
        out_shape=jax.ShapeDtypeStruct((N_BLK, 1, 128), jnp.float32),
        compiler_params=pltpu.CompilerParams(
            dimension_semantics=("arbitrary",)),
    )(pxy2, txy2, mask2)


def kernel(output, mask, ind, target):
    output4 = output.reshape(B, C, HW // 128, 128)
    pred = _gather(output4, ind.reshape(B, 1, K), ind.reshape(B, K, 1))
    pxy2 = pred.reshape(P2, 2 * C)
    txy2 = target.reshape(P2, 2 * C)
    partials = _polyloss(pxy2, txy2, mask.reshape(P2, 2))
    s_iou = jnp.sum(partials[:, 0, 0])
    s_msk = jnp.sum(partials[:, 0, 1])
    return 1.0 - s_iou / (s_msk + 1e-6)


# speedup vs baseline: 1.1080x; 1.0059x over previous
import functools

import jax
import jax.numpy as jnp
from jax.experimental import pallas as pl
from jax.experimental.pallas import tpu as pltpu

V = 16
MAXN = 4 * V
B, K, H, W = 16, 128, 256, 256
C = 2 * V
HW = H * W
KG = 16
N_POLY = B * K
P2 = N_POLY // 2
P2_BLK = 128
N_BLK = P2 // P2_BLK


def _gather_body(x_ref, iw_ref, out_ref):
    f32 = jnp.float32
    iw = iw_ref[0] // 128
    ir = iw_ref[0] % 128
    sub512 = jax.lax.broadcasted_iota(jnp.int32, (HW // 128, K), 0)
    ow = (sub512 == iw).astype(f32)
    sub128 = jax.lax.broadcasted_iota(jnp.int32, (128, K), 0)
    ohr = (sub128 == ir).astype(f32)
    rows = []
    for c in range(C):
        w1t = jax.lax.dot_general(
            x_ref[0, c], ow, (((0,), (0,)), ((), ())),
            preferred_element_type=f32)
        rows.append(jnp.sum(w1t * ohr, axis=0, keepdims=True))
    out_ref[0] = jnp.concatenate(rows, axis=0)


def _gather(output4, ind3):
    return pl.pallas_call(
        _gather_body,
        grid=(B,),
        in_specs=[
            pl.BlockSpec((1, C, HW // 128, 128), lambda b: (b, 0, 0, 0)),
            pl.BlockSpec((1, 1, K), lambda b: (b, 0, 0)),
        ],
        out_specs=pl.BlockSpec((1, C, K), lambda b: (b, 0, 0)),
        out_shape=jax.ShapeDtypeStruct((B, C, K), jnp.float32),
        compiler_params=pltpu.CompilerParams(
            dimension_semantics=("arbitrary",)),
    )(output4, ind3)


def _shr1(a):
    return jnp.concatenate(
        [jnp.zeros((a.shape[0], 1), a.dtype), a[:, :-1]], axis=1)


def _shrk(a, k):
    return jnp.concatenate(
        [jnp.zeros((a.shape[0], k), a.dtype), a[:, :-k]], axis=1)


def _shl1(a):
    return jnp.concatenate(
        [a[:, 1:], jnp.zeros((a.shape[0], 1), a.dtype)], axis=1)


def _safe(d):
    return jnp.where(d == 0.0, jnp.ones_like(d), d)


def _shoelace_pair(x32, y32):
    xn = jnp.concatenate(
        [x32[:, 1:16], x32[:, :1], x32[:, 17:32], x32[:, 16:17]], axis=1)
    yn = jnp.concatenate(
        [y32[:, 1:16], y32[:, :1], y32[:, 17:32], y32[:, 16:17]], axis=1)
    tl = x32 * yn
    tr = y32 * xn
    lL = jnp.sum(tl[:, :16], axis=1, keepdims=True) + x32[:, :1] * y32[:, 1:2]
    rL = jnp.sum(tr[:, :16], axis=1, keepdims=True) + y32[:, :1] * x32[:, 1:2]
    lR = jnp.sum(tl[:, 16:], axis=1, keepdims=True) + x32[:, 16:17] * y32[:, 17:18]
    rR = jnp.sum(tr[:, 16:], axis=1, keepdims=True) + y32[:, 16:17] * x32[:, 17:18]
    return jnp.abs(0.5 * (rL - lL)), jnp.abs(0.5 * (rR - lR))


def _polyloss_body(pxy_ref, txy_ref, m_ref, out_ref):
    f32 = jnp.float32
    i16 = jax.lax.broadcasted_iota(jnp.int32, (P2_BLK, V), 1)
    a = pxy_ref[...]
    t = txy_ref[...]
    px2 = jnp.take_along_axis(a, 2 * i16, axis=1)
    py2 = jnp.take_along_axis(a, 2 * i16 + 1, axis=1)
    qx2 = jnp.take_along_axis(a, 2 * i16 + 2 * V, axis=1)
    qy2 = jnp.take_along_axis(a, 2 * i16 + 2 * V + 1, axis=1)
    tx2 = jnp.take_along_axis(t, 2 * i16, axis=1)
    ty2 = jnp.take_along_axis(t, 2 * i16 + 1, axis=1)
    ux2 = jnp.take_along_axis(t, 2 * i16 + 2 * V, axis=1)
    uy2 = jnp.take_along_axis(t, 2 * i16 + 2 * V + 1, axis=1)

    P = P2_BLK
    ii = jax.lax.broadcasted_iota(jnp.int32, (P, 2 * MAXN), 1)
    half = ii >= MAXN
    iota = ii.astype(f32)
    iotam = jnp.where(half, iota - float(MAXN), iota)
    zpad = jnp.zeros((P, MAXN - V), f32)
    x = jnp.concatenate([px2, zpad, qx2, zpad], axis=1)
    y = jnp.concatenate([py2, zpad, qy2, zpad], axis=1)
    nL = jnp.full((P, 1), float(V), f32)
    nR = jnp.full((P, 1), float(V), f32)
    base = jnp.where(half, MAXN, 0)

    for e in range(V):
        ep = (e - 1) % V
        c1x = jnp.where(half, ux2[:, ep:ep + 1], tx2[:, ep:ep + 1])
        c1y = jnp.where(half, uy2[:, ep:ep + 1], ty2[:, ep:ep + 1])
        c2x = jnp.where(half, ux2[:, e:e + 1], tx2[:, e:e + 1])
        c2y = jnp.where(half, uy2[:, e:e + 1], ty2[:, e:e + 1])
        n_b = jnp.where(half, nR, nL)

        idxlast = jnp.concatenate(
            [jnp.maximum(nL.astype(jnp.int32) - 1, 0),
             jnp.maximum(nR.astype(jnp.int32) - 1, 0) + MAXN], axis=1)
        lx = jnp.take_along_axis(x, idxlast, axis=1)
        ly = jnp.take_along_axis(y, idxlast, axis=1)
        pxv = jnp.where(ii == 0, lx[:, 0:1],
                        jnp.where(ii == MAXN, lx[:, 1:2], _shr1(x)))
        pyv = jnp.where(ii == 0, ly[:, 0:1],
                        jnp.where(ii == MAXN, ly[:, 1:2], _shr1(y)))

        valid = iotam < n_b
        ex, ey = c2x - c1x, c2y - c1y
        ins_c = (ex * (y - c1y) - ey * (x - c1x)) <= 0.0
        ins_p = (ex * (pyv - c1y) - ey * (pxv - c1x)) <= 0.0

        dx12 = x - pxv
        dy12 = y - pyv
        m1 = dy12 / _safe(dx12)
        b1 = pyv - m1 * pxv
        m2 = ey / _safe(ex)
        b2 = c1y - m2 * c1x
        x_gen = (b2 - b1) / _safe(m1 - m2)
        y_gen = m1 * x_gen + b1
        y_v1 = m2 * pxv + b2
        y_v2 = m1 * c1x + b1
        vert1 = dx12 == 0.0
        vert2 = ex == 0.0
        ix = jnp.where(vert1, pxv, jnp.where(vert2, c1x, x_gen))
        iy = jnp.where(vert1, y_v1, jnp.where(vert2, y_v2, y_gen))

        emit_i = valid & (ins_c != ins_p)
        emit_c = valid & ins_c
        cnt = emit_i.astype(f32) + emit_c.astype(f32)
        csum = cnt
        for s in (1, 2, 4, 8, 16, 32, 64):
            csum = csum + _shrk(csum, s)
        csum = jnp.where(half, csum - csum[:, MAXN - 1:MAXN], csum)
        totL = csum[:, MAXN - 1:MAXN]
        totR = csum[:, 2 * MAXN - 1:2 * MAXN]
        new_nL = jnp.minimum(totL, float(MAXN))
        new_nR = jnp.minimum(totR, float(MAXN))
        new_nb = jnp.where(half, new_nR, new_nL)

        lo = base
        for bit in (32, 16, 8, 4, 2, 1):
            cand = lo + bit
            cs = jnp.take_along_axis(csum, cand - 1, axis=1)
            lo = jnp.where(cs <= iotam, cand, lo)

        lo_prev = jnp.concatenate(
            [jnp.full((P, 1), -1, jnp.int32), lo[:, :2 * MAXN - 1]], axis=1)
        is_first = lo > lo_prev
        fx = jnp.where(emit_i, ix, x)
        fy = jnp.where(emit_i, iy, y)
        g_fx = jnp.take_along_axis(fx, lo, axis=1)
        g_fy = jnp.take_along_axis(fy, lo, axis=1)
        g_cx = jnp.take_along_axis(x, lo, axis=1)
        g_cy = jnp.take_along_axis(y, lo, axis=1)
        keep = iotam < new_nb
        x = jnp.where(keep, jnp.where(is_first, g_fx, g_cx), 0.0)
        y = jnp.where(keep, jnp.where(is_first, g_fy, g_cy), 0.0)
        nL, nR = new_nL, new_nR

    n_b = jnp.where(half, nR, nL)
    valid_f = (iotam < n_b).astype(f32)
    is_last = iotam == (n_b - 1.0)
    x0b = jnp.where(half, x[:, MAXN:MAXN + 1], x[:, :1])
    y0b = jnp.where(half, y[:, MAXN:MAXN + 1], y[:, :1])
    x_nxt = jnp.where(is_last, x0b, _shl1(x))
    y_nxt = jnp.where(is_last, y0b, _shl1(y))
    tl = valid_f * x * y_nxt
    tr = valid_f * y * x_nxt
    hf = jnp.where(half, 1.0, 0.0).astype(f32)
    lL = jnp.sum(tl * (1.0 - hf), axis=1, keepdims=True)
    lR = jnp.sum(tl * hf, axis=1, keepdims=True)
    rL = jnp.sum(tr * (1.0 - hf), axis=1, keepdims=True)
    rR = jnp.sum(tr * hf, axis=1, keepdims=True)
    hasL = (nL > 0.0).astype(f32)
    hasR = (nR > 0.0).astype(f32)
    y1L = jnp.where(nL == 1.0, y[:, :1], y[:, 1:2])
    x1L = jnp.where(nL == 1.0, x[:, :1], x[:, 1:2])
    y1R = jnp.where(nR == 1.0, y[:, MAXN:MAXN + 1], y[:, MAXN + 1:MAXN + 2])
    x1R = jnp.where(nR == 1.0, x[:, MAXN:MAXN + 1], x[:, MAXN + 1:MAXN + 2])
    a_iL = jnp.abs(0.5 * ((rL + hasL * y[:, :1] * x1L) - (lL + hasL * x[:, :1] * y1L)))
    a_iR = jnp.abs(0.5 * ((rR + hasR * y[:, MAXN:MAXN + 1] * x1R)
                          - (lR + hasR * x[:, MAXN:MAXN + 1] * y1R)))

    x32p = jnp.concatenate([px2, qx2], axis=1)
    y32p = jnp.concatenate([py2, qy2], axis=1)
    x32t = jnp.concatenate([tx2, ux2], axis=1)
    y32t = jnp.concatenate([ty2, uy2], axis=1)
    a_pL, a_pR = _shoelace_pair(x32p, y32p)
    a_gL, a_gR = _shoelace_pair(x32t, y32t)

    intL = jnp.where(a_iL == 0.0, jnp.minimum(a_pL, a_gL), 0.0) + a_iL
    intR = jnp.where(a_iR == 0.0, jnp.minimum(a_pR, a_gR), 0.0) + a_iR
    iouL = intL / (a_gL + a_pL - intL + 1e-6)
    iouR = intR / (a_gR + a_pR - intR + 1e-6)

    s_iou = jnp.sum(iouL * m_ref[:, 0:1] + iouR * m_ref[:, 1:2])
    s_msk = jnp.sum(m_ref[...])
    li = jax.lax.broadcasted_iota(jnp.int32, (1, 128), 1)
    out_ref[0] = jnp.where(li == 0, s_iou, jnp.where(li == 1, s_msk, 0.0))


def _polyloss(pxy2, txy2, mask2):
    spec64 = pl.BlockSpec((P2_BLK, 4 * V), lambda g: (g, 0))
    spec2 = pl.BlockSpec((P2_BLK, 2), lambda g: (g, 0))
    return pl.pallas_call(
        _polyloss_body,
        grid=(N_BLK,),
        in_specs=[spec64, spec64, spec2],
        out_specs=pl.BlockSpec((1, 1, 128), lambda g: (g, 0, 0)),
        out_shape=jax.ShapeDtypeStruct((N_BLK, 1, 128), jnp.float32),
        compiler_params=pltpu.CompilerParams(
            dimension_semantics=("arbitrary",)),
    )(pxy2, txy2, mask2)


def kernel(output, mask, ind, target):
    output4 = output.reshape(B, C, HW // 128, 128)
    predt = _gather(output4, ind.reshape(B, 1, K))
    pred = predt.transpose(0, 2, 1).reshape(N_POLY, C)
    pxy2 = pred.reshape(P2, 2 * C)
    txy2 = target.reshape(P2, 2 * C)
    partials = _polyloss(pxy2, txy2, mask.reshape(P2, 2))
    s_iou = jnp.sum(partials[:, 0, 0])
    s_msk = jnp.sum(partials[:, 0, 1])
    return 1.0 - s_iou / (s_msk + 1e-6)
